# Initial kernel scaffold; baseline (speedup 1.0000x reference)
import jax
import jax.numpy as jnp
from jax.experimental import pallas as pl


def kernel(userEmbedding, itemEmbedding, ratingEmbedding, Wa_item, ba_item, Wo_item, bo_item, Ws, bs, Wa_user, ba_user, Wo_user, bo_user, W2, b2, u_idx, i_idx, r_idx, t_src, t_dst):
    raise NotImplementedError("write your pallas kernel here")


# speedup vs baseline: 7.3670x; 7.3670x over previous
import jax
import jax.numpy as jnp
from jax import lax
from jax.experimental import pallas as pl
from jax.experimental.pallas import tpu as pltpu
from jax.experimental.pallas import tpu_sc as plsc

F32 = jnp.float32

_U = 50000
_I = 50000
_D = 32
_R = 6
_NS = 16
_NC = 2
_STRIPE = 3128
_PADN = _STRIPE * _NS
_UNITS_E = 6400
_UNITS_T = 6400


def _relu(x):
    return jnp.maximum(x, 0.0)


def _q_body(rE, WaI, baI, WaU, baU, qi, qu):
    r = rE[...]
    qi[...] = jnp.dot(r, WaI[_D:, :], preferred_element_type=F32) + baI[...]
    qu[...] = jnp.dot(r, WaU[_D:, :], preferred_element_type=F32) + baU[...]


def _table_body(item, user, WaI, WaU, qi, qu, xi, xu):
    r = pl.program_id(1)
    pi = jnp.dot(item[...], WaI[:_D, :], preferred_element_type=F32)
    pu = jnp.dot(user[...], WaU[:_D, :], preferred_element_type=F32)
    xi[...] = _relu(pi + qi[pl.ds(r, 1), :])
    xu[...] = _relu(pu + qu[pl.ds(r, 1), :])


def _gidx_body(r2, ig, ug, gi, gu):
    r = r2[...]
    gi[...] = r * _I + ig[...]
    gu[...] = r * _U + ug[...]


def _head_body(su, cu, si, ci, WoI, boI, WoU, boU, hI, z):
    mu = su[...] / jnp.maximum(cu[...], 1.0)
    mi = si[...] / jnp.maximum(ci[...], 1.0)
    hI[...] = _relu(jnp.dot(mu, WoI[...], preferred_element_type=F32) + boI[...])
    z[...] = _relu(jnp.dot(mi, WoU[...], preferred_element_type=F32) + boU[...])


def _final_body(t0, t1, c0, c1, hI, Ws, bs, W2, b2, h):
    t = t0[...] + t1[...]
    c = jnp.maximum(c0[...] + c1[...], 1.0)
    hS = _relu(jnp.dot(t / c, Ws[...], preferred_element_type=F32) + bs[...])
    h[...] = _relu(jnp.dot(hI[...], W2[:_D, :], preferred_element_type=F32)
                   + jnp.dot(hS, W2[_D:, :], preferred_element_type=F32) + b2[...])


def _fill_ones(ones):
    for v in range(ones.shape[0] // 16):
        ones[pl.ds(v * 16, 16)] = jnp.ones((16,), F32)


def _fill_zeros(buf):
    for v in range(buf.shape[0] // 16):
        buf[pl.ds(v * 16, 16)] = jnp.zeros((16,), F32)


def _sc_accumulate(G, S, X, out_sum, out_cnt, zrows, cbuf,
                   gidx, sidx, rows, ones, sem, acc, cnt,
                   sid, unit0, n_chunks, K):
    row0 = sid * _STRIPE
    pltpu.sync_copy(zrows, acc.at[pl.ds(row0, _STRIPE)])
    pltpu.sync_copy(cbuf.at[pl.ds(0, _STRIPE)], cnt.at[pl.ds(row0, _STRIPE)])
    plsc.subcore_barrier()

    def chunk(k, carry):
        u0 = unit0 + k * K
        pltpu.sync_copy(G.at[pl.ds(u0, K)], gidx)
        pltpu.sync_copy(S.at[pl.ds(u0, K)], sidx)
        for j in range(K):
            pltpu.async_copy(X.at[gidx.at[j]], rows, sem).wait()
            pltpu.sync_copy(rows, acc.at[sidx.at[j]], add=True)
            pltpu.sync_copy(ones, cnt.at[sidx.at[j]], add=True)
        return carry

    lax.fori_loop(0, n_chunks, chunk, 0)
    plsc.subcore_barrier()
    pltpu.sync_copy(acc.at[pl.ds(row0, _STRIPE)], out_sum.at[pl.ds(row0, _STRIPE)])
    pltpu.sync_copy(cnt.at[pl.ds(row0, _STRIPE)], cbuf.at[pl.ds(0, _STRIPE)])
    pltpu.sync_copy(cbuf.at[pl.ds(0, _STRIPE)], out_cnt.at[pl.ds(row0, _STRIPE)])


def _edge_kernel(gitem, guser, us_, is_, xitem, xuser, zrows,
                 su, cu, si, ci, gidx, sidx, rows, ones, cbuf, sem, acc, cnt):
    c = lax.axis_index("c")
    s = lax.axis_index("s")
    _fill_ones(ones)
    _fill_zeros(cbuf)
    per_tile = _UNITS_E // _NS
    unit0 = s * per_tile

    def item_side():
        _sc_accumulate(gitem, us_, xitem, su, cu, zrows, cbuf,
                       gidx, sidx, rows, ones, sem, acc, cnt,
                       s, unit0, per_tile // 16, 16)

    def user_side():
        _sc_accumulate(guser, is_, xuser, si, ci, zrows, cbuf,
                       gidx, sidx, rows, ones, sem, acc, cnt,
                       s, unit0, per_tile // 16, 16)

    pl.when(c == 0)(item_side)
    pl.when(c == 1)(user_side)


def _trust_kernel(tsrc, tdst, hI, zrows,
                  t0, c0, t1, c1, gidx, sidx, rows, ones, cbuf, sem, acc, cnt):
    c = lax.axis_index("c")
    s = lax.axis_index("s")
    _fill_ones(ones)
    _fill_zeros(cbuf)
    per_core = _UNITS_T // _NC
    per_tile = per_core // _NS
    unit0 = c * per_core + s * per_tile

    def core0():
        _sc_accumulate(tsrc, tdst, hI, t0, c0, zrows, cbuf,
                       gidx, sidx, rows, ones, sem, acc, cnt,
                       s, unit0, per_tile // 8, 8)

    def core1():
        _sc_accumulate(tsrc, tdst, hI, t1, c1, zrows, cbuf,
                       gidx, sidx, rows, ones, sem, acc, cnt,
                       s, unit0, per_tile // 8, 8)

    pl.when(c == 0)(core0)
    pl.when(c == 1)(core1)


def kernel(userEmbedding, itemEmbedding, ratingEmbedding, Wa_item, ba_item,
           Wo_item, bo_item, Ws, bs, Wa_user, ba_user, Wo_user, bo_user,
           W2, b2, u_idx, i_idx, r_idx, t_src, t_dst):
    E = u_idx.shape[0]
    ET = t_src.shape[0]

    qi, qu = pl.pallas_call(
        _q_body,
        out_shape=[jax.ShapeDtypeStruct((_R, _D), F32)] * 2,
    )(ratingEmbedding, Wa_item, ba_item.reshape(1, _D),
      Wa_user, ba_user.reshape(1, _D))

    BT = 1000
    xi, xu = pl.pallas_call(
        _table_body,
        grid=(_U // BT, _R),
        in_specs=[
            pl.BlockSpec((BT, _D), lambda i, r: (i, 0)),
            pl.BlockSpec((BT, _D), lambda i, r: (i, 0)),
            pl.BlockSpec((2 * _D, _D), lambda i, r: (0, 0)),
            pl.BlockSpec((2 * _D, _D), lambda i, r: (0, 0)),
            pl.BlockSpec((_R, _D), lambda i, r: (0, 0)),
            pl.BlockSpec((_R, _D), lambda i, r: (0, 0)),
        ],
        out_specs=[
            pl.BlockSpec((BT, _D), lambda i, r: (r * 50 + i, 0)),
            pl.BlockSpec((BT, _D), lambda i, r: (r * 50 + i, 0)),
        ],
        out_shape=[jax.ShapeDtypeStruct((_R * _U, _D), F32)] * 2,
    )(itemEmbedding, userEmbedding, Wa_item, Wa_user, qi, qu)

    pad_e = _UNITS_E * 128 - E
    z32 = jnp.zeros((pad_e,), jnp.int32)
    r_p = jnp.concatenate([r_idx, z32]).reshape(_UNITS_E, 128)
    ig = jnp.concatenate([i_idx, z32]).reshape(_UNITS_E, 128)
    ug = jnp.concatenate([u_idx, z32]).reshape(_UNITS_E, 128)
    us_ = jnp.concatenate([u_idx, jnp.full((pad_e,), _U, jnp.int32)]).reshape(_UNITS_E, 128)
    is_ = jnp.concatenate([i_idx, jnp.full((pad_e,), _I, jnp.int32)]).reshape(_UNITS_E, 128)

    gitem, guser = pl.pallas_call(
        _gidx_body,
        grid=(10,),
        in_specs=[pl.BlockSpec((_UNITS_E // 10, 128), lambda i: (i, 0))] * 3,
        out_specs=[pl.BlockSpec((_UNITS_E // 10, 128), lambda i: (i, 0))] * 2,
        out_shape=[jax.ShapeDtypeStruct((_UNITS_E, 128), jnp.int32)] * 2,
    )(r_p, ig, ug)

    zrows = jnp.zeros((_STRIPE, _D), F32)

    mesh = plsc.VectorSubcoreMesh(core_axis_name="c", subcore_axis_name="s")
    su, cu, si, ci = pl.kernel(
        _edge_kernel,
        out_type=[
            jax.ShapeDtypeStruct((_PADN, _D), F32),
            jax.ShapeDtypeStruct((_PADN,), F32),
            jax.ShapeDtypeStruct((_PADN, _D), F32),
            jax.ShapeDtypeStruct((_PADN,), F32),
        ],
        mesh=mesh,
        compiler_params=pltpu.CompilerParams(use_tc_tiling_on_sc=False),
        scratch_types=[
            pltpu.VMEM((16, 128), jnp.int32),
            pltpu.VMEM((16, 128), jnp.int32),
            pltpu.VMEM((128, _D), F32),
            pltpu.VMEM((128,), F32),
            pltpu.VMEM((_STRIPE + 8, ), F32),
            pltpu.SemaphoreType.DMA,
            pltpu.VMEM_SHARED((_PADN, _D), F32),
            pltpu.VMEM_SHARED((_PADN,), F32),
        ],
    )(gitem, guser, us_, is_, xi, xu, zrows)

    BH = _STRIPE
    hI, z = pl.pallas_call(
        _head_body,
        grid=(_NS,),
        in_specs=[
            pl.BlockSpec((BH, _D), lambda i: (i, 0)),
            pl.BlockSpec((BH, 1), lambda i: (i, 0)),
            pl.BlockSpec((BH, _D), lambda i: (i, 0)),
            pl.BlockSpec((BH, 1), lambda i: (i, 0)),
            pl.BlockSpec((_D, _D), lambda i: (0, 0)),
            pl.BlockSpec((1, _D), lambda i: (0, 0)),
            pl.BlockSpec((_D, _D), lambda i: (0, 0)),
            pl.BlockSpec((1, _D), lambda i: (0, 0)),
        ],
        out_specs=[pl.BlockSpec((BH, _D), lambda i: (i, 0))] * 2,
        out_shape=[jax.ShapeDtypeStruct((_PADN, _D), F32)] * 2,
    )(su, cu.reshape(_PADN, 1), si, ci.reshape(_PADN, 1),
      Wo_item, bo_item.reshape(1, _D), Wo_user, bo_user.reshape(1, _D))

    pad_t = _UNITS_T * 128 - ET
    tsrc_p = jnp.concatenate([t_src, jnp.zeros((pad_t,), jnp.int32)]).reshape(_UNITS_T, 128)
    tdst_p = jnp.concatenate([t_dst, jnp.full((pad_t,), _U, jnp.int32)]).reshape(_UNITS_T, 128)

    t0, c0, t1, c1 = pl.kernel(
        _trust_kernel,
        out_type=[
            jax.ShapeDtypeStruct((_PADN, _D), F32),
            jax.ShapeDtypeStruct((_PADN,), F32),
            jax.ShapeDtypeStruct((_PADN, _D), F32),
            jax.ShapeDtypeStruct((_PADN,), F32),
        ],
        mesh=mesh,
        compiler_params=pltpu.CompilerParams(use_tc_tiling_on_sc=False),
        scratch_types=[
            pltpu.VMEM((8, 128), jnp.int32),
            pltpu.VMEM((8, 128), jnp.int32),
            pltpu.VMEM((128, _D), F32),
            pltpu.VMEM((128,), F32),
            pltpu.VMEM((_STRIPE + 8, ), F32),
            pltpu.SemaphoreType.DMA,
            pltpu.VMEM_SHARED((_PADN, _D), F32),
            pltpu.VMEM_SHARED((_PADN,), F32),
        ],
    )(tsrc_p, tdst_p, hI, zrows)

    h = pl.pallas_call(
        _final_body,
        grid=(_NS,),
        in_specs=[
            pl.BlockSpec((BH, _D), lambda i: (i, 0)),
            pl.BlockSpec((BH, _D), lambda i: (i, 0)),
            pl.BlockSpec((BH, 1), lambda i: (i, 0)),
            pl.BlockSpec((BH, 1), lambda i: (i, 0)),
            pl.BlockSpec((BH, _D), lambda i: (i, 0)),
            pl.BlockSpec((_D, _D), lambda i: (0, 0)),
            pl.BlockSpec((1, _D), lambda i: (0, 0)),
            pl.BlockSpec((2 * _D, _D), lambda i: (0, 0)),
            pl.BlockSpec((1, _D), lambda i: (0, 0)),
        ],
        out_specs=pl.BlockSpec((BH, _D), lambda i: (i, 0)),
        out_shape=jax.ShapeDtypeStruct((_PADN, _D), F32),
    )(t0, t1, c0.reshape(_PADN, 1), c1.reshape(_PADN, 1), hI,
      Ws, bs.reshape(1, _D), W2, b2.reshape(1, _D))

    return (h[:_U], z[:_U])


# speedup vs baseline: 8.2037x; 1.1136x over previous
import jax
import jax.numpy as jnp
from jax import lax
from jax.experimental import pallas as pl
from jax.experimental.pallas import tpu as pltpu
from jax.experimental.pallas import tpu_sc as plsc

F32 = jnp.float32

_U = 50000
_I = 50000
_D = 32
_R = 6
_NS = 16
_NC = 2
_STRIPE = 3128
_PADN = _STRIPE * _NS
_UNITS_E = 6400
_UNITS_T = 6400


def _relu(x):
    return jnp.maximum(x, 0.0)


def _q_body(rE, WaI, baI, WaU, baU, qi, qu):
    r = rE[...]
    qi[...] = jnp.dot(r, WaI[_D:, :], preferred_element_type=F32) + baI[...]
    qu[...] = jnp.dot(r, WaU[_D:, :], preferred_element_type=F32) + baU[...]


def _table_body(item, user, WaI, WaU, qi, qu, xi, xu):
    r = pl.program_id(1)
    pi = jnp.dot(item[...], WaI[:_D, :], preferred_element_type=F32)
    pu = jnp.dot(user[...], WaU[:_D, :], preferred_element_type=F32)
    xi[...] = _relu(pi + qi[pl.ds(r, 1), :])
    xu[...] = _relu(pu + qu[pl.ds(r, 1), :])


def _gidx_body(r2, ig, ug, gi, gu):
    r = r2[...]
    gi[...] = r * _I + ig[...]
    gu[...] = r * _U + ug[...]


def _head_body(su, cu, si, ci, WoI, boI, WoU, boU, hI, z):
    mu = su[...] / jnp.maximum(cu[...], 1.0)
    mi = si[...] / jnp.maximum(ci[...], 1.0)
    hI[...] = _relu(jnp.dot(mu, WoI[...], preferred_element_type=F32) + boI[...])
    z[...] = _relu(jnp.dot(mi, WoU[...], preferred_element_type=F32) + boU[...])


def _final_body(t0, t1, c0, c1, hI, Ws, bs, W2, b2, h):
    t = t0[...] + t1[...]
    c = jnp.maximum(c0[...] + c1[...], 1.0)
    hS = _relu(jnp.dot(t / c, Ws[...], preferred_element_type=F32) + bs[...])
    h[...] = _relu(jnp.dot(hI[...], W2[:_D, :], preferred_element_type=F32)
                   + jnp.dot(hS, W2[_D:, :], preferred_element_type=F32) + b2[...])


def _fill_ones(ones):
    for v in range(ones.shape[0] // 16):
        ones[pl.ds(v * 16, 16)] = jnp.ones((16,), F32)


def _fill_zeros(buf):
    for v in range(buf.shape[0] // 16):
        buf[pl.ds(v * 16, 16)] = jnp.zeros((16,), F32)


def _sc_accumulate(G, S, X, out_sum, out_cnt, zrows, cbuf,
                   gidx, sidx, rows, ones, gsem, ssem, osem, acc, cnt,
                   sid, unit0, n_chunks, K):
    row0 = sid * _STRIPE
    pltpu.sync_copy(zrows, acc.at[pl.ds(row0, _STRIPE)])
    pltpu.sync_copy(cbuf.at[pl.ds(0, _STRIPE)], cnt.at[pl.ds(row0, _STRIPE)])
    plsc.subcore_barrier()

    def chunk(k, carry):
        u0 = unit0 + k * K
        pltpu.sync_copy(G.at[pl.ds(u0, K)], gidx)
        pltpu.sync_copy(S.at[pl.ds(u0, K)], sidx)
        gd = [pltpu.async_copy(X.at[gidx.at[j]],
                               rows.at[pl.ds(j * 128, 128)], gsem.at[j])
              for j in range(K)]
        sd = []
        for j in range(K):
            gd[j].wait()
            sd.append(pltpu.async_copy(rows.at[pl.ds(j * 128, 128)],
                                       acc.at[sidx.at[j]], ssem, add=True))
            sd.append(pltpu.async_copy(ones, cnt.at[sidx.at[j]], osem, add=True))
        for d in sd:
            d.wait()
        return carry

    lax.fori_loop(0, n_chunks, chunk, 0)
    plsc.subcore_barrier()
    pltpu.sync_copy(acc.at[pl.ds(row0, _STRIPE)], out_sum.at[pl.ds(row0, _STRIPE)])
    pltpu.sync_copy(cnt.at[pl.ds(row0, _STRIPE)], cbuf.at[pl.ds(0, _STRIPE)])
    pltpu.sync_copy(cbuf.at[pl.ds(0, _STRIPE)], out_cnt.at[pl.ds(row0, _STRIPE)])


_K = 4


def _edge_kernel(gitem, guser, us_, is_, xitem, xuser, zrows,
                 su, cu, si, ci, gidx, sidx, rows, ones, cbuf,
                 gsem, ssem, osem, acc, cnt):
    c = lax.axis_index("c")
    s = lax.axis_index("s")
    _fill_ones(ones)
    _fill_zeros(cbuf)
    per_tile = _UNITS_E // _NS
    unit0 = s * per_tile

    def item_side():
        _sc_accumulate(gitem, us_, xitem, su, cu, zrows, cbuf,
                       gidx, sidx, rows, ones, gsem, ssem, osem, acc, cnt,
                       s, unit0, per_tile // _K, _K)

    def user_side():
        _sc_accumulate(guser, is_, xuser, si, ci, zrows, cbuf,
                       gidx, sidx, rows, ones, gsem, ssem, osem, acc, cnt,
                       s, unit0, per_tile // _K, _K)

    pl.when(c == 0)(item_side)
    pl.when(c == 1)(user_side)


def _trust_kernel(tsrc, tdst, hI, zrows,
                  t0, c0, t1, c1, gidx, sidx, rows, ones, cbuf,
                  gsem, ssem, osem, acc, cnt):
    c = lax.axis_index("c")
    s = lax.axis_index("s")
    _fill_ones(ones)
    _fill_zeros(cbuf)
    per_core = _UNITS_T // _NC
    per_tile = per_core // _NS
    unit0 = c * per_core + s * per_tile

    def core0():
        _sc_accumulate(tsrc, tdst, hI, t0, c0, zrows, cbuf,
                       gidx, sidx, rows, ones, gsem, ssem, osem, acc, cnt,
                       s, unit0, per_tile // _K, _K)

    def core1():
        _sc_accumulate(tsrc, tdst, hI, t1, c1, zrows, cbuf,
                       gidx, sidx, rows, ones, gsem, ssem, osem, acc, cnt,
                       s, unit0, per_tile // _K, _K)

    pl.when(c == 0)(core0)
    pl.when(c == 1)(core1)


def kernel(userEmbedding, itemEmbedding, ratingEmbedding, Wa_item, ba_item,
           Wo_item, bo_item, Ws, bs, Wa_user, ba_user, Wo_user, bo_user,
           W2, b2, u_idx, i_idx, r_idx, t_src, t_dst):
    E = u_idx.shape[0]
    ET = t_src.shape[0]

    qi, qu = pl.pallas_call(
        _q_body,
        out_shape=[jax.ShapeDtypeStruct((_R, _D), F32)] * 2,
    )(ratingEmbedding, Wa_item, ba_item.reshape(1, _D),
      Wa_user, ba_user.reshape(1, _D))

    BT = 1000
    xi, xu = pl.pallas_call(
        _table_body,
        grid=(_U // BT, _R),
        in_specs=[
            pl.BlockSpec((BT, _D), lambda i, r: (i, 0)),
            pl.BlockSpec((BT, _D), lambda i, r: (i, 0)),
            pl.BlockSpec((2 * _D, _D), lambda i, r: (0, 0)),
            pl.BlockSpec((2 * _D, _D), lambda i, r: (0, 0)),
            pl.BlockSpec((_R, _D), lambda i, r: (0, 0)),
            pl.BlockSpec((_R, _D), lambda i, r: (0, 0)),
        ],
        out_specs=[
            pl.BlockSpec((BT, _D), lambda i, r: (r * 50 + i, 0)),
            pl.BlockSpec((BT, _D), lambda i, r: (r * 50 + i, 0)),
        ],
        out_shape=[jax.ShapeDtypeStruct((_R * _U, _D), F32)] * 2,
    )(itemEmbedding, userEmbedding, Wa_item, Wa_user, qi, qu)

    pad_e = _UNITS_E * 128 - E
    z32 = jnp.zeros((pad_e,), jnp.int32)
    r_p = jnp.concatenate([r_idx, z32]).reshape(_UNITS_E, 128)
    ig = jnp.concatenate([i_idx, z32]).reshape(_UNITS_E, 128)
    ug = jnp.concatenate([u_idx, z32]).reshape(_UNITS_E, 128)
    us_ = jnp.concatenate([u_idx, jnp.full((pad_e,), _U, jnp.int32)]).reshape(_UNITS_E, 128)
    is_ = jnp.concatenate([i_idx, jnp.full((pad_e,), _I, jnp.int32)]).reshape(_UNITS_E, 128)

    gitem, guser = pl.pallas_call(
        _gidx_body,
        grid=(10,),
        in_specs=[pl.BlockSpec((_UNITS_E // 10, 128), lambda i: (i, 0))] * 3,
        out_specs=[pl.BlockSpec((_UNITS_E // 10, 128), lambda i: (i, 0))] * 2,
        out_shape=[jax.ShapeDtypeStruct((_UNITS_E, 128), jnp.int32)] * 2,
    )(r_p, ig, ug)

    zrows = jnp.zeros((_STRIPE, _D), F32)

    mesh = plsc.VectorSubcoreMesh(core_axis_name="c", subcore_axis_name="s")
    su, cu, si, ci = pl.kernel(
        _edge_kernel,
        out_type=[
            jax.ShapeDtypeStruct((_PADN, _D), F32),
            jax.ShapeDtypeStruct((_PADN,), F32),
            jax.ShapeDtypeStruct((_PADN, _D), F32),
            jax.ShapeDtypeStruct((_PADN,), F32),
        ],
        mesh=mesh,
        compiler_params=pltpu.CompilerParams(use_tc_tiling_on_sc=False),
        scratch_types=[
            pltpu.VMEM((_K, 128), jnp.int32),
            pltpu.VMEM((_K, 128), jnp.int32),
            pltpu.VMEM((_K * 128, _D), F32),
            pltpu.VMEM((128,), F32),
            pltpu.VMEM((_STRIPE + 8, ), F32),
            pltpu.SemaphoreType.DMA((_K,)),
            pltpu.SemaphoreType.DMA,
            pltpu.SemaphoreType.DMA,
            pltpu.VMEM_SHARED((_PADN, _D), F32),
            pltpu.VMEM_SHARED((_PADN,), F32),
        ],
    )(gitem, guser, us_, is_, xi, xu, zrows)

    BH = _STRIPE
    hI, z = pl.pallas_call(
        _head_body,
        grid=(_NS,),
        in_specs=[
            pl.BlockSpec((BH, _D), lambda i: (i, 0)),
            pl.BlockSpec((BH, 1), lambda i: (i, 0)),
            pl.BlockSpec((BH, _D), lambda i: (i, 0)),
            pl.BlockSpec((BH, 1), lambda i: (i, 0)),
            pl.BlockSpec((_D, _D), lambda i: (0, 0)),
            pl.BlockSpec((1, _D), lambda i: (0, 0)),
            pl.BlockSpec((_D, _D), lambda i: (0, 0)),
            pl.BlockSpec((1, _D), lambda i: (0, 0)),
        ],
        out_specs=[pl.BlockSpec((BH, _D), lambda i: (i, 0))] * 2,
        out_shape=[jax.ShapeDtypeStruct((_PADN, _D), F32)] * 2,
    )(su, cu.reshape(_PADN, 1), si, ci.reshape(_PADN, 1),
      Wo_item, bo_item.reshape(1, _D), Wo_user, bo_user.reshape(1, _D))

    pad_t = _UNITS_T * 128 - ET
    tsrc_p = jnp.concatenate([t_src, jnp.zeros((pad_t,), jnp.int32)]).reshape(_UNITS_T, 128)
    tdst_p = jnp.concatenate([t_dst, jnp.full((pad_t,), _U, jnp.int32)]).reshape(_UNITS_T, 128)

    t0, c0, t1, c1 = pl.kernel(
        _trust_kernel,
        out_type=[
            jax.ShapeDtypeStruct((_PADN, _D), F32),
            jax.ShapeDtypeStruct((_PADN,), F32),
            jax.ShapeDtypeStruct((_PADN, _D), F32),
            jax.ShapeDtypeStruct((_PADN,), F32),
        ],
        mesh=mesh,
        compiler_params=pltpu.CompilerParams(use_tc_tiling_on_sc=False),
        scratch_types=[
            pltpu.VMEM((_K, 128), jnp.int32),
            pltpu.VMEM((_K, 128), jnp.int32),
            pltpu.VMEM((_K * 128, _D), F32),
            pltpu.VMEM((128,), F32),
            pltpu.VMEM((_STRIPE + 8, ), F32),
            pltpu.SemaphoreType.DMA((_K,)),
            pltpu.SemaphoreType.DMA,
            pltpu.SemaphoreType.DMA,
            pltpu.VMEM_SHARED((_PADN, _D), F32),
            pltpu.VMEM_SHARED((_PADN,), F32),
        ],
    )(tsrc_p, tdst_p, hI, zrows)

    h = pl.pallas_call(
        _final_body,
        grid=(_NS,),
        in_specs=[
            pl.BlockSpec((BH, _D), lambda i: (i, 0)),
            pl.BlockSpec((BH, _D), lambda i: (i, 0)),
            pl.BlockSpec((BH, 1), lambda i: (i, 0)),
            pl.BlockSpec((BH, 1), lambda i: (i, 0)),
            pl.BlockSpec((BH, _D), lambda i: (i, 0)),
            pl.BlockSpec((_D, _D), lambda i: (0, 0)),
            pl.BlockSpec((1, _D), lambda i: (0, 0)),
            pl.BlockSpec((2 * _D, _D), lambda i: (0, 0)),
            pl.BlockSpec((1, _D), lambda i: (0, 0)),
        ],
        out_specs=pl.BlockSpec((BH, _D), lambda i: (i, 0)),
        out_shape=jax.ShapeDtypeStruct((_PADN, _D), F32),
    )(t0, t1, c0.reshape(_PADN, 1), c1.reshape(_PADN, 1), hI,
      Ws, bs.reshape(1, _D), W2, b2.reshape(1, _D))

    return (h[:_U], z[:_U])


# speedup vs baseline: 8.5904x; 1.0471x over previous
import jax
import jax.numpy as jnp
from jax import lax
from jax.experimental import pallas as pl
from jax.experimental.pallas import tpu as pltpu
from jax.experimental.pallas import tpu_sc as plsc

F32 = jnp.float32

_U = 50000
_I = 50000
_D = 32
_R = 6
_NS = 16
_NC = 2
_STRIPE = 3128
_PADN = _STRIPE * _NS
_UNITS_E = 6400
_UNITS_T = 6400
_K = 5


def _relu(x):
    return jnp.maximum(x, 0.0)


def _q_body(rE, WaI, baI, WaU, baU, qi, qu):
    r = rE[...]
    qi[...] = jnp.dot(r, WaI[_D:, :], preferred_element_type=F32) + baI[...]
    qu[...] = jnp.dot(r, WaU[_D:, :], preferred_element_type=F32) + baU[...]


def _table_body(item, user, WaI, WaU, qi, qu, xi, xu):
    r = pl.program_id(1)
    pi = jnp.dot(item[...], WaI[:_D, :], preferred_element_type=F32)
    pu = jnp.dot(user[...], WaU[:_D, :], preferred_element_type=F32)
    xi[...] = _relu(pi + qi[pl.ds(r, 1), :])
    xu[...] = _relu(pu + qu[pl.ds(r, 1), :])


def _gidx_body(r2, ig, ug, gi, gu):
    r = r2[...]
    gi[...] = r * _I + ig[...]
    gu[...] = r * _U + ug[...]


def _head_body(su, cu, si, ci, WoI, boI, WoU, boU, hI, z):
    mu = su[...] / jnp.maximum(cu[...], 1.0)
    mi = si[...] / jnp.maximum(ci[...], 1.0)
    hI[...] = _relu(jnp.dot(mu, WoI[...], preferred_element_type=F32) + boI[...])
    z[...] = _relu(jnp.dot(mi, WoU[...], preferred_element_type=F32) + boU[...])


def _final_body(t0, t1, c0, c1, hI, Ws, bs, W2, b2, h):
    t = t0[...] + t1[...]
    c = jnp.maximum(c0[...] + c1[...], 1.0)
    hS = _relu(jnp.dot(t / c, Ws[...], preferred_element_type=F32) + bs[...])
    h[...] = _relu(jnp.dot(hI[...], W2[:_D, :], preferred_element_type=F32)
                   + jnp.dot(hS, W2[_D:, :], preferred_element_type=F32) + b2[...])


def _fill_ones(ones):
    for v in range(ones.shape[0] // 16):
        ones[pl.ds(v * 16, 16)] = jnp.ones((16,), F32)


def _fill_zeros(buf):
    for v in range(buf.shape[0] // 16):
        buf[pl.ds(v * 16, 16)] = jnp.zeros((16,), F32)


def _sc_accumulate(G, S, X, out_sum, out_cnt, zrows, cbuf,
                   gidx, sidx, rows, ones, gsem, ssem, osem, acc, cnt,
                   sid, unit0, n_chunks, K):
    row0 = sid * _STRIPE
    pltpu.sync_copy(zrows, acc.at[pl.ds(row0, _STRIPE)])
    pltpu.sync_copy(cbuf.at[pl.ds(0, _STRIPE)], cnt.at[pl.ds(row0, _STRIPE)])
    plsc.subcore_barrier()

    def chunk(k, carry):
        u0 = unit0 + k * K
        pltpu.sync_copy(G.at[pl.ds(u0, K)], gidx)
        pltpu.sync_copy(S.at[pl.ds(u0, K)], sidx)
        gd = [pltpu.async_copy(X.at[gidx.at[j]],
                               rows.at[pl.ds(j * 128, 128)], gsem.at[j])
              for j in range(K)]
        sd = []
        for j in range(K):
            gd[j].wait()
            sd.append(pltpu.async_copy(rows.at[pl.ds(j * 128, 128)],
                                       acc.at[sidx.at[j]], ssem, add=True))
            sd.append(pltpu.async_copy(ones, cnt.at[sidx.at[j]], osem, add=True))
        for d in sd:
            d.wait()
        return carry

    lax.fori_loop(0, n_chunks, chunk, 0)
    plsc.subcore_barrier()
    pltpu.sync_copy(acc.at[pl.ds(row0, _STRIPE)], out_sum.at[pl.ds(row0, _STRIPE)])
    pltpu.sync_copy(cnt.at[pl.ds(row0, _STRIPE)], cbuf.at[pl.ds(0, _STRIPE)])
    pltpu.sync_copy(cbuf.at[pl.ds(0, _STRIPE)], out_cnt.at[pl.ds(row0, _STRIPE)])


def _edge_kernel(gitem, guser, us_, is_, xitem, xuser, zrows,
                 su, cu, si, ci, gidx, sidx, rows, ones, cbuf,
                 gsem, ssem, osem, acc, cnt):
    c = lax.axis_index("c")
    s = lax.axis_index("s")
    _fill_ones(ones)
    _fill_zeros(cbuf)
    per_tile = _UNITS_E // _NS
    unit0 = s * per_tile

    def item_side():
        _sc_accumulate(gitem, us_, xitem, su, cu, zrows, cbuf,
                       gidx, sidx, rows, ones, gsem, ssem, osem, acc, cnt,
                       s, unit0, per_tile // _K, _K)

    def user_side():
        _sc_accumulate(guser, is_, xuser, si, ci, zrows, cbuf,
                       gidx, sidx, rows, ones, gsem, ssem, osem, acc, cnt,
                       s, unit0, per_tile // _K, _K)

    pl.when(c == 0)(item_side)
    pl.when(c == 1)(user_side)


def _trust_kernel(tsrc, tdst, hI, zrows,
                  t0, c0, t1, c1, gidx, sidx, rows, ones, cbuf,
                  gsem, ssem, osem, acc, cnt):
    c = lax.axis_index("c")
    s = lax.axis_index("s")
    _fill_ones(ones)
    _fill_zeros(cbuf)
    per_core = _UNITS_T // _NC
    per_tile = per_core // _NS
    unit0 = c * per_core + s * per_tile

    def core0():
        _sc_accumulate(tsrc, tdst, hI, t0, c0, zrows, cbuf,
                       gidx, sidx, rows, ones, gsem, ssem, osem, acc, cnt,
                       s, unit0, per_tile // _K, _K)

    def core1():
        _sc_accumulate(tsrc, tdst, hI, t1, c1, zrows, cbuf,
                       gidx, sidx, rows, ones, gsem, ssem, osem, acc, cnt,
                       s, unit0, per_tile // _K, _K)

    pl.when(c == 0)(core0)
    pl.when(c == 1)(core1)


def kernel(userEmbedding, itemEmbedding, ratingEmbedding, Wa_item, ba_item,
           Wo_item, bo_item, Ws, bs, Wa_user, ba_user, Wo_user, bo_user,
           W2, b2, u_idx, i_idx, r_idx, t_src, t_dst):
    E = u_idx.shape[0]
    ET = t_src.shape[0]

    qi, qu = pl.pallas_call(
        _q_body,
        out_shape=[jax.ShapeDtypeStruct((_R, _D), F32)] * 2,
    )(ratingEmbedding, Wa_item, ba_item.reshape(1, _D),
      Wa_user, ba_user.reshape(1, _D))

    BT = 1000
    xi, xu = pl.pallas_call(
        _table_body,
        grid=(_U // BT, _R),
        in_specs=[
            pl.BlockSpec((BT, _D), lambda i, r: (i, 0)),
            pl.BlockSpec((BT, _D), lambda i, r: (i, 0)),
            pl.BlockSpec((2 * _D, _D), lambda i, r: (0, 0)),
            pl.BlockSpec((2 * _D, _D), lambda i, r: (0, 0)),
            pl.BlockSpec((_R, _D), lambda i, r: (0, 0)),
            pl.BlockSpec((_R, _D), lambda i, r: (0, 0)),
        ],
        out_specs=[
            pl.BlockSpec((BT, _D), lambda i, r: (r * 50 + i, 0)),
            pl.BlockSpec((BT, _D), lambda i, r: (r * 50 + i, 0)),
        ],
        out_shape=[jax.ShapeDtypeStruct((_R * _U, _D), F32)] * 2,
    )(itemEmbedding, userEmbedding, Wa_item, Wa_user, qi, qu)

    pad_e = _UNITS_E * 128 - E
    z32 = jnp.zeros((pad_e,), jnp.int32)
    r_p = jnp.concatenate([r_idx, z32]).reshape(_UNITS_E, 128)
    ig = jnp.concatenate([i_idx, z32]).reshape(_UNITS_E, 128)
    ug = jnp.concatenate([u_idx, z32]).reshape(_UNITS_E, 128)
    us_ = jnp.concatenate([u_idx, jnp.full((pad_e,), _U, jnp.int32)]).reshape(_UNITS_E, 128)
    is_ = jnp.concatenate([i_idx, jnp.full((pad_e,), _I, jnp.int32)]).reshape(_UNITS_E, 128)

    gitem, guser = pl.pallas_call(
        _gidx_body,
        grid=(10,),
        in_specs=[pl.BlockSpec((_UNITS_E // 10, 128), lambda i: (i, 0))] * 3,
        out_specs=[pl.BlockSpec((_UNITS_E // 10, 128), lambda i: (i, 0))] * 2,
        out_shape=[jax.ShapeDtypeStruct((_UNITS_E, 128), jnp.int32)] * 2,
    )(r_p, ig, ug)

    zrows = jnp.zeros((_STRIPE, _D), F32)

    def sc_scratch():
        return [
            pltpu.VMEM((_K, 128), jnp.int32),
            pltpu.VMEM((_K, 128), jnp.int32),
            pltpu.VMEM((_K * 128, _D), F32),
            pltpu.VMEM((128,), F32),
            pltpu.VMEM((_STRIPE + 8, ), F32),
            pltpu.SemaphoreType.DMA((_K,)),
            pltpu.SemaphoreType.DMA,
            pltpu.SemaphoreType.DMA,
            pltpu.VMEM_SHARED((_PADN, _D), F32),
            pltpu.VMEM_SHARED((_PADN,), F32),
        ]

    mesh = plsc.VectorSubcoreMesh(core_axis_name="c", subcore_axis_name="s")
    su, cu, si, ci = pl.kernel(
        _edge_kernel,
        out_type=[
            jax.ShapeDtypeStruct((_PADN, _D), F32),
            jax.ShapeDtypeStruct((_PADN,), F32),
            jax.ShapeDtypeStruct((_PADN, _D), F32),
            jax.ShapeDtypeStruct((_PADN,), F32),
        ],
        mesh=mesh,
        compiler_params=pltpu.CompilerParams(use_tc_tiling_on_sc=False),
        scratch_types=sc_scratch(),
    )(gitem, guser, us_, is_, xi, xu, zrows)

    BH = _STRIPE
    hI, z = pl.pallas_call(
        _head_body,
        grid=(_NS,),
        in_specs=[
            pl.BlockSpec((BH, _D), lambda i: (i, 0)),
            pl.BlockSpec((BH, 1), lambda i: (i, 0)),
            pl.BlockSpec((BH, _D), lambda i: (i, 0)),
            pl.BlockSpec((BH, 1), lambda i: (i, 0)),
            pl.BlockSpec((_D, _D), lambda i: (0, 0)),
            pl.BlockSpec((1, _D), lambda i: (0, 0)),
            pl.BlockSpec((_D, _D), lambda i: (0, 0)),
            pl.BlockSpec((1, _D), lambda i: (0, 0)),
        ],
        out_specs=[pl.BlockSpec((BH, _D), lambda i: (i, 0))] * 2,
        out_shape=[jax.ShapeDtypeStruct((_PADN, _D), F32)] * 2,
    )(su, cu.reshape(_PADN, 1), si, ci.reshape(_PADN, 1),
      Wo_item, bo_item.reshape(1, _D), Wo_user, bo_user.reshape(1, _D))

    pad_t = _UNITS_T * 128 - ET
    tsrc_p = jnp.concatenate([t_src, jnp.zeros((pad_t,), jnp.int32)]).reshape(_UNITS_T, 128)
    tdst_p = jnp.concatenate([t_dst, jnp.full((pad_t,), _U, jnp.int32)]).reshape(_UNITS_T, 128)

    t0, c0, t1, c1 = pl.kernel(
        _trust_kernel,
        out_type=[
            jax.ShapeDtypeStruct((_PADN, _D), F32),
            jax.ShapeDtypeStruct((_PADN,), F32),
            jax.ShapeDtypeStruct((_PADN, _D), F32),
            jax.ShapeDtypeStruct((_PADN,), F32),
        ],
        mesh=mesh,
        compiler_params=pltpu.CompilerParams(use_tc_tiling_on_sc=False),
        scratch_types=sc_scratch(),
    )(tsrc_p, tdst_p, hI, zrows)

    h = pl.pallas_call(
        _final_body,
        grid=(_NS,),
        in_specs=[
            pl.BlockSpec((BH, _D), lambda i: (i, 0)),
            pl.BlockSpec((BH, _D), lambda i: (i, 0)),
            pl.BlockSpec((BH, 1), lambda i: (i, 0)),
            pl.BlockSpec((BH, 1), lambda i: (i, 0)),
            pl.BlockSpec((BH, _D), lambda i: (i, 0)),
            pl.BlockSpec((_D, _D), lambda i: (0, 0)),
            pl.BlockSpec((1, _D), lambda i: (0, 0)),
            pl.BlockSpec((2 * _D, _D), lambda i: (0, 0)),
            pl.BlockSpec((1, _D), lambda i: (0, 0)),
        ],
        out_specs=pl.BlockSpec((BH, _D), lambda i: (i, 0)),
        out_shape=jax.ShapeDtypeStruct((_PADN, _D), F32),
    )(t0, t1, c0.reshape(_PADN, 1), c1.reshape(_PADN, 1), hI,
      Ws, bs.reshape(1, _D), W2, b2.reshape(1, _D))

    return (h[:_U], z[:_U])


# speedup vs baseline: 9.4785x; 1.1034x over previous
import jax
import jax.numpy as jnp
from jax import lax
from jax.experimental import pallas as pl
from jax.experimental.pallas import tpu as pltpu
from jax.experimental.pallas import tpu_sc as plsc

F32 = jnp.float32

_U = 50000
_I = 50000
_D = 32
_R = 6
_NS = 16
_NC = 2
_STRIPE = 3128
_PADN = _STRIPE * _NS
_UNITS_E = 6400
_UNITS_T = 6400
_K = 5


def _relu(x):
    return jnp.maximum(x, 0.0)


def _q_body(rE, WaI, baI, WaU, baU, qi, qu):
    r = rE[...]
    qi[...] = jnp.dot(r, WaI[_D:, :], preferred_element_type=F32) + baI[...]
    qu[...] = jnp.dot(r, WaU[_D:, :], preferred_element_type=F32) + baU[...]


def _table_body(item, user, WaI, WaU, qi, qu, xi, xu):
    r = pl.program_id(1)
    pi = jnp.dot(item[...], WaI[:_D, :], preferred_element_type=F32)
    pu = jnp.dot(user[...], WaU[:_D, :], preferred_element_type=F32)
    xi[...] = _relu(pi + qi[pl.ds(r, 1), :])
    xu[...] = _relu(pu + qu[pl.ds(r, 1), :])


def _gidx_body(r2, ig, ug, gi, gu):
    r = r2[...]
    gi[...] = r * _I + ig[...]
    gu[...] = r * _U + ug[...]


def _head_body(su, cu, si, ci, WoI, boI, WoU, boU, hI, z):
    mu = su[...] / jnp.maximum(cu[...], 1.0)
    mi = si[...] / jnp.maximum(ci[...], 1.0)
    hI[...] = _relu(jnp.dot(mu, WoI[...], preferred_element_type=F32) + boI[...])
    z[...] = _relu(jnp.dot(mi, WoU[...], preferred_element_type=F32) + boU[...])


def _final_body(t0, t1, c0, c1, hI, Ws, bs, W2, b2, h):
    t = t0[...] + t1[...]
    c = jnp.maximum(c0[...] + c1[...], 1.0)
    hS = _relu(jnp.dot(t / c, Ws[...], preferred_element_type=F32) + bs[...])
    h[...] = _relu(jnp.dot(hI[...], W2[:_D, :], preferred_element_type=F32)
                   + jnp.dot(hS, W2[_D:, :], preferred_element_type=F32) + b2[...])


def _fill_ones(ones):
    for v in range(ones.shape[0] // 16):
        ones[pl.ds(v * 16, 16)] = jnp.ones((16,), F32)


def _fill_zeros(buf):
    for v in range(buf.shape[0] // 16):
        buf[pl.ds(v * 16, 16)] = jnp.zeros((16,), F32)


def _sc_accumulate(GS, X, out_sum, out_cnt, zrows, cbuf,
                   idxa, idxb, rows, ones, gsem, ssem, osem, isem, acc, cnt,
                   sid, unit0, n_chunks, K):
    row0 = sid * _STRIPE
    pltpu.sync_copy(zrows, acc.at[pl.ds(row0, _STRIPE)])
    pltpu.sync_copy(cbuf.at[pl.ds(0, _STRIPE)], cnt.at[pl.ds(row0, _STRIPE)])
    plsc.subcore_barrier()

    def run_chunk(idx):
        gd = [pltpu.async_copy(X.at[idx.at[j, 0]],
                               rows.at[pl.ds(j * 128, 128)], gsem.at[j])
              for j in range(K)]
        sd = []
        for j in range(K):
            gd[j].wait()
            sd.append(pltpu.async_copy(rows.at[pl.ds(j * 128, 128)],
                                       acc.at[idx.at[j, 1]], ssem, add=True))
            sd.append(pltpu.async_copy(ones, cnt.at[idx.at[j, 1]], osem, add=True))
        for d in sd:
            d.wait()

    pltpu.sync_copy(GS.at[pl.ds(unit0, K)], idxa)

    def pair(k2, carry):
        u0 = unit0 + 2 * k2 * K
        db = pltpu.async_copy(GS.at[pl.ds(u0 + K, K)], idxb, isem)
        run_chunk(idxa)
        db.wait()

        @pl.when(k2 + 1 < n_chunks // 2)
        def _prefetch():
            da = pltpu.async_copy(GS.at[pl.ds(u0 + 2 * K, K)], idxa, isem)
            run_chunk(idxb)
            da.wait()

        @pl.when(k2 + 1 >= n_chunks // 2)
        def _last():
            run_chunk(idxb)

        return carry

    lax.fori_loop(0, n_chunks // 2, pair, 0)
    plsc.subcore_barrier()
    pltpu.sync_copy(acc.at[pl.ds(row0, _STRIPE)], out_sum.at[pl.ds(row0, _STRIPE)])
    pltpu.sync_copy(cnt.at[pl.ds(row0, _STRIPE)], cbuf.at[pl.ds(0, _STRIPE)])
    pltpu.sync_copy(cbuf.at[pl.ds(0, _STRIPE)], out_cnt.at[pl.ds(row0, _STRIPE)])


def _edge_kernel(gsitem, gsuser, xitem, xuser, zrows,
                 su, cu, si, ci, idxa, idxb, rows, ones, cbuf,
                 gsem, ssem, osem, isem, acc, cnt):
    c = lax.axis_index("c")
    s = lax.axis_index("s")
    _fill_ones(ones)
    _fill_zeros(cbuf)
    per_tile = _UNITS_E // _NS
    unit0 = s * per_tile

    def item_side():
        _sc_accumulate(gsitem, xitem, su, cu, zrows, cbuf,
                       idxa, idxb, rows, ones, gsem, ssem, osem, isem, acc, cnt,
                       s, unit0, per_tile // _K, _K)

    def user_side():
        _sc_accumulate(gsuser, xuser, si, ci, zrows, cbuf,
                       idxa, idxb, rows, ones, gsem, ssem, osem, isem, acc, cnt,
                       s, unit0, per_tile // _K, _K)

    pl.when(c == 0)(item_side)
    pl.when(c == 1)(user_side)


def _trust_kernel(gst, hI, zrows,
                  t0, c0, t1, c1, idxa, idxb, rows, ones, cbuf,
                  gsem, ssem, osem, isem, acc, cnt):
    c = lax.axis_index("c")
    s = lax.axis_index("s")
    _fill_ones(ones)
    _fill_zeros(cbuf)
    per_core = _UNITS_T // _NC
    per_tile = per_core // _NS
    unit0 = c * per_core + s * per_tile

    def core0():
        _sc_accumulate(gst, hI, t0, c0, zrows, cbuf,
                       idxa, idxb, rows, ones, gsem, ssem, osem, isem, acc, cnt,
                       s, unit0, per_tile // _K, _K)

    def core1():
        _sc_accumulate(gst, hI, t1, c1, zrows, cbuf,
                       idxa, idxb, rows, ones, gsem, ssem, osem, isem, acc, cnt,
                       s, unit0, per_tile // _K, _K)

    pl.when(c == 0)(core0)
    pl.when(c == 1)(core1)


def kernel(userEmbedding, itemEmbedding, ratingEmbedding, Wa_item, ba_item,
           Wo_item, bo_item, Ws, bs, Wa_user, ba_user, Wo_user, bo_user,
           W2, b2, u_idx, i_idx, r_idx, t_src, t_dst):
    E = u_idx.shape[0]
    ET = t_src.shape[0]

    qi, qu = pl.pallas_call(
        _q_body,
        out_shape=[jax.ShapeDtypeStruct((_R, _D), F32)] * 2,
    )(ratingEmbedding, Wa_item, ba_item.reshape(1, _D),
      Wa_user, ba_user.reshape(1, _D))

    BT = 1000
    xi, xu = pl.pallas_call(
        _table_body,
        grid=(_U // BT, _R),
        in_specs=[
            pl.BlockSpec((BT, _D), lambda i, r: (i, 0)),
            pl.BlockSpec((BT, _D), lambda i, r: (i, 0)),
            pl.BlockSpec((2 * _D, _D), lambda i, r: (0, 0)),
            pl.BlockSpec((2 * _D, _D), lambda i, r: (0, 0)),
            pl.BlockSpec((_R, _D), lambda i, r: (0, 0)),
            pl.BlockSpec((_R, _D), lambda i, r: (0, 0)),
        ],
        out_specs=[
            pl.BlockSpec((BT, _D), lambda i, r: (r * 50 + i, 0)),
            pl.BlockSpec((BT, _D), lambda i, r: (r * 50 + i, 0)),
        ],
        out_shape=[jax.ShapeDtypeStruct((_R * _U, _D), F32)] * 2,
    )(itemEmbedding, userEmbedding, Wa_item, Wa_user, qi, qu)

    pad_e = _UNITS_E * 128 - E
    z32 = jnp.zeros((pad_e,), jnp.int32)
    r_p = jnp.concatenate([r_idx, z32]).reshape(_UNITS_E, 128)
    ig = jnp.concatenate([i_idx, z32]).reshape(_UNITS_E, 128)
    ug = jnp.concatenate([u_idx, z32]).reshape(_UNITS_E, 128)
    us_ = jnp.concatenate([u_idx, jnp.full((pad_e,), _U, jnp.int32)]).reshape(_UNITS_E, 128)
    is_ = jnp.concatenate([i_idx, jnp.full((pad_e,), _I, jnp.int32)]).reshape(_UNITS_E, 128)

    gitem, guser = pl.pallas_call(
        _gidx_body,
        grid=(10,),
        in_specs=[pl.BlockSpec((_UNITS_E // 10, 128), lambda i: (i, 0))] * 3,
        out_specs=[pl.BlockSpec((_UNITS_E // 10, 128), lambda i: (i, 0))] * 2,
        out_shape=[jax.ShapeDtypeStruct((_UNITS_E, 128), jnp.int32)] * 2,
    )(r_p, ig, ug)

    zrows = jnp.zeros((_STRIPE, _D), F32)

    def sc_scratch():
        return [
            pltpu.VMEM((_K, 2, 128), jnp.int32),
            pltpu.VMEM((_K, 2, 128), jnp.int32),
            pltpu.VMEM((_K * 128, _D), F32),
            pltpu.VMEM((128,), F32),
            pltpu.VMEM((_STRIPE + 8, ), F32),
            pltpu.SemaphoreType.DMA((_K,)),
            pltpu.SemaphoreType.DMA,
            pltpu.SemaphoreType.DMA,
            pltpu.SemaphoreType.DMA,
            pltpu.VMEM_SHARED((_PADN, _D), F32),
            pltpu.VMEM_SHARED((_PADN,), F32),
        ]

    mesh = plsc.VectorSubcoreMesh(core_axis_name="c", subcore_axis_name="s")
    su, cu, si, ci = pl.kernel(
        _edge_kernel,
        out_type=[
            jax.ShapeDtypeStruct((_PADN, _D), F32),
            jax.ShapeDtypeStruct((_PADN,), F32),
            jax.ShapeDtypeStruct((_PADN, _D), F32),
            jax.ShapeDtypeStruct((_PADN,), F32),
        ],
        mesh=mesh,
        compiler_params=pltpu.CompilerParams(use_tc_tiling_on_sc=False),
        scratch_types=sc_scratch(),
    )(jnp.stack([gitem, us_], axis=1), jnp.stack([guser, is_], axis=1),
      xi, xu, zrows)

    BH = _STRIPE
    hI, z = pl.pallas_call(
        _head_body,
        grid=(_NS,),
        in_specs=[
            pl.BlockSpec((BH, _D), lambda i: (i, 0)),
            pl.BlockSpec((BH, 1), lambda i: (i, 0)),
            pl.BlockSpec((BH, _D), lambda i: (i, 0)),
            pl.BlockSpec((BH, 1), lambda i: (i, 0)),
            pl.BlockSpec((_D, _D), lambda i: (0, 0)),
            pl.BlockSpec((1, _D), lambda i: (0, 0)),
            pl.BlockSpec((_D, _D), lambda i: (0, 0)),
            pl.BlockSpec((1, _D), lambda i: (0, 0)),
        ],
        out_specs=[pl.BlockSpec((BH, _D), lambda i: (i, 0))] * 2,
        out_shape=[jax.ShapeDtypeStruct((_PADN, _D), F32)] * 2,
    )(su, cu.reshape(_PADN, 1), si, ci.reshape(_PADN, 1),
      Wo_item, bo_item.reshape(1, _D), Wo_user, bo_user.reshape(1, _D))

    pad_t = _UNITS_T * 128 - ET
    tsrc_p = jnp.concatenate([t_src, jnp.zeros((pad_t,), jnp.int32)]).reshape(_UNITS_T, 128)
    tdst_p = jnp.concatenate([t_dst, jnp.full((pad_t,), _U, jnp.int32)]).reshape(_UNITS_T, 128)

    t0, c0, t1, c1 = pl.kernel(
        _trust_kernel,
        out_type=[
            jax.ShapeDtypeStruct((_PADN, _D), F32),
            jax.ShapeDtypeStruct((_PADN,), F32),
            jax.ShapeDtypeStruct((_PADN, _D), F32),
            jax.ShapeDtypeStruct((_PADN,), F32),
        ],
        mesh=mesh,
        compiler_params=pltpu.CompilerParams(use_tc_tiling_on_sc=False),
        scratch_types=sc_scratch(),
    )(jnp.stack([tsrc_p, tdst_p], axis=1), hI, zrows)

    h = pl.pallas_call(
        _final_body,
        grid=(_NS,),
        in_specs=[
            pl.BlockSpec((BH, _D), lambda i: (i, 0)),
            pl.BlockSpec((BH, _D), lambda i: (i, 0)),
            pl.BlockSpec((BH, 1), lambda i: (i, 0)),
            pl.BlockSpec((BH, 1), lambda i: (i, 0)),
            pl.BlockSpec((BH, _D), lambda i: (i, 0)),
            pl.BlockSpec((_D, _D), lambda i: (0, 0)),
            pl.BlockSpec((1, _D), lambda i: (0, 0)),
            pl.BlockSpec((2 * _D, _D), lambda i: (0, 0)),
            pl.BlockSpec((1, _D), lambda i: (0, 0)),
        ],
        out_specs=pl.BlockSpec((BH, _D), lambda i: (i, 0)),
        out_shape=jax.ShapeDtypeStruct((_PADN, _D), F32),
    )(t0, t1, c0.reshape(_PADN, 1), c1.reshape(_PADN, 1), hI,
      Ws, bs.reshape(1, _D), W2, b2.reshape(1, _D))

    return (h[:_U], z[:_U])
